# baseline (speedup 1.0000x reference)
import functools

import jax
import jax.numpy as jnp
from jax import lax
from jax.experimental import pallas as pl
from jax.experimental.pallas import tpu as pltpu
from jax.experimental.pallas import tpu_sc as plsc

N = 10000
E = 320000
F_IN = 128
H = 16
C = 40
INV_BN = 1.0 / (1.0 + 1e-5) ** 0.5

NC = 2
NS = 16
L = 16
NW = NC * NS
EPT = E // NW
K = 80
NCH = EPT // K
NBUF = 5
RND = NCH // NBUF
ROWS_T = 624
TAIL = N - ROWS_T * NS

_GDN = lax.GatherDimensionNumbers(
    offset_dims=(), collapsed_slice_dims=(0,), start_index_map=(0,))


def _lane_bcast(v, r):
    idx = jnp.full((L, 1), r, dtype=jnp.int32)
    return lax.gather(v, idx, _GDN, (1,),
                      mode=lax.GatherScatterMode.PROMISE_IN_BOUNDS)


_MESH = plsc.VectorSubcoreMesh(core_axis_name="c", subcore_axis_name="s")

_EDGE_SCRATCH = [
    pltpu.VMEM((EPT,), jnp.int32),
    pltpu.VMEM((EPT,), jnp.int32),
    pltpu.VMEM((EPT,), jnp.float32),
    pltpu.VMEM_SHARED((N, H), jnp.float32),
]
_RING_SCRATCH = ([pltpu.VMEM((K, L), jnp.float32)] * (2 * NBUF)
                 + [pltpu.SemaphoreType.DMA] * (2 * NBUF))


def _stage_edges(ei_hbm, w_hbm, z_hbm, src_v, dst_v, w_v, acc, c, s):
    ebase0 = (c * NS + s) * EPT
    pltpu.sync_copy(ei_hbm.at[0, pl.ds(ebase0, EPT)], src_v)
    pltpu.sync_copy(ei_hbm.at[1, pl.ds(ebase0, EPT)], dst_v)
    pltpu.sync_copy(w_hbm.at[pl.ds(ebase0, EPT)], w_v)
    rbase = s * ROWS_T
    pltpu.sync_copy(z_hbm.at[pl.ds(rbase, ROWS_T)],
                    acc.at[pl.ds(rbase, ROWS_T)])

    @pl.when(s == 0)
    def _zero_tail():
        pltpu.sync_copy(z_hbm.at[pl.ds(ROWS_T * NS, TAIL)],
                        acc.at[pl.ds(ROWS_T * NS, TAIL)])


def _emit_spmm(ysrc, src_v, dst_v, w_v, acc, scr):
    gbuf = scr[0:NBUF]
    sbuf = scr[NBUF:2 * NBUF]
    gsem = scr[2 * NBUF:3 * NBUF]
    ssem = scr[3 * NBUF:4 * NBUF]
    iota16 = lax.iota(jnp.int32, L)

    def g_start(jj, b):
        pltpu.async_copy(ysrc.at[src_v.at[pl.ds(jj * K, K)]],
                         gbuf[b], gsem[b])

    def g_wait(b):
        pltpu.make_async_copy(ysrc.at[src_v.at[pl.ds(0, K)]],
                              gbuf[b], gsem[b]).wait()

    def s_start(jj, b):
        for q in range(K // L):
            idx16 = dst_v[pl.ds(jj * K + q * L, L)]
            pltpu.async_copy(sbuf[b].at[pl.ds(q * L, L)], acc.at[idx16],
                             ssem[b], add=True)

    def s_wait(b):
        for q in range(K // L):
            pltpu.make_async_copy(sbuf[b].at[pl.ds(q * L, L)],
                                  acc.at[iota16], ssem[b]).wait()

    def compute(jj, b):
        for q in range(K // L):
            w16 = w_v[pl.ds(jj * K + q * L, L)]
            for r in range(L):
                e = q * L + r
                sbuf[b][e] = gbuf[b][e] * _lane_bcast(w16, r)

    for b in range(NBUF):
        g_start(b, b)

    def round_body(rr, carry):
        for b in range(NBUF):
            jj = rr * NBUF + b
            g_wait(b)

            @pl.when(rr > 0)
            def _drain():
                s_wait(b)

            compute(jj, b)
            s_start(jj, b)

            @pl.when(rr < RND - 1)
            def _prefetch():
                g_start(jj + NBUF, b)

        return carry

    lax.fori_loop(0, RND, round_body, 0)
    for b in range(NBUF):
        s_wait(b)


def _write_out(acc, out_hbm, c, s):
    rbase = s * ROWS_T
    pltpu.sync_copy(acc.at[pl.ds(rbase, ROWS_T)],
                    out_hbm.at[c, pl.ds(rbase, ROWS_T)])

    @pl.when(s == 0)
    def _write_tail():
        pltpu.sync_copy(acc.at[pl.ds(ROWS_T * NS, TAIL)],
                        out_hbm.at[c, pl.ds(ROWS_T * NS, TAIL)])


def _spmm_sc(y, ei, w, zeros):
    @functools.partial(
        pl.kernel,
        mesh=_MESH,
        out_type=jax.ShapeDtypeStruct((NC, N, H), jnp.float32),
        scratch_types=_EDGE_SCRATCH + _RING_SCRATCH,
        compiler_params=pltpu.CompilerParams(use_tc_tiling_on_sc=False),
    )
    def k(y_hbm, ei_hbm, w_hbm, z_hbm, out_hbm,
          src_v, dst_v, w_v, acc, *scr):
        c = lax.axis_index("c")
        s = lax.axis_index("s")
        _stage_edges(ei_hbm, w_hbm, z_hbm, src_v, dst_v, w_v,
                     acc, c, s)
        plsc.subcore_barrier()
        _emit_spmm(y_hbm, src_v, dst_v, w_v, acc, scr)
        plsc.subcore_barrier()
        _write_out(acc, out_hbm, c, s)

    return k(y, ei, w, zeros)


def _post0_rows(pa_v, pb_v, zz_v, hbuf_v, bias, gi, be, n_rows):
    def grp(g, carry):
        z16 = zz_v[pl.ds(g * L, L)]
        for r in range(L):
            i = g * L + r
            t = (pa_v[i] + pb_v[i]) * _lane_bcast(z16, r) + bias
            hbuf_v[i] = jnp.maximum(t * gi + be, 0.0)
        return carry

    lax.fori_loop(0, n_rows // L, grp, 0)


def _spmm_post0_sc(p, z, aff, ei, w, zeros):
    @functools.partial(
        pl.kernel,
        mesh=_MESH,
        out_type=(jax.ShapeDtypeStruct((NC, N, H), jnp.float32),
                  jax.ShapeDtypeStruct((NC, N, H), jnp.float32)),
        scratch_types=(_EDGE_SCRATCH + [
            pltpu.VMEM((ROWS_T, H), jnp.float32),
            pltpu.VMEM((ROWS_T, H), jnp.float32),
            pltpu.VMEM((ROWS_T,), jnp.float32),
            pltpu.VMEM((ROWS_T, H), jnp.float32),
            pltpu.VMEM((3, H), jnp.float32),
        ] + _RING_SCRATCH),
        compiler_params=pltpu.CompilerParams(use_tc_tiling_on_sc=False),
    )
    def k(p_hbm, z_hbm, aff_hbm, ei_hbm, w_hbm, zz_hbm, out_hbm, h_hbm,
          src_v, dst_v, w_v, acc, pa_v, pb_v, znode_v, hbuf_v, aff_v,
          *scr):
        c = lax.axis_index("c")
        s = lax.axis_index("s")
        _stage_edges(ei_hbm, w_hbm, zz_hbm, src_v, dst_v, w_v,
                     acc, c, s)
        rbase = s * ROWS_T
        pltpu.sync_copy(p_hbm.at[0, pl.ds(rbase, ROWS_T)], pa_v)
        pltpu.sync_copy(p_hbm.at[1, pl.ds(rbase, ROWS_T)], pb_v)
        pltpu.sync_copy(z_hbm.at[pl.ds(rbase, ROWS_T)], znode_v)
        pltpu.sync_copy(aff_hbm, aff_v)
        bias, gi, be = aff_v[0], aff_v[1], aff_v[2]
        _post0_rows(pa_v, pb_v, znode_v, hbuf_v, bias, gi, be, ROWS_T)
        pltpu.sync_copy(hbuf_v, h_hbm.at[c, pl.ds(rbase, ROWS_T)])

        @pl.when(s == 0)
        def _tail():
            tb = ROWS_T * NS
            pltpu.sync_copy(p_hbm.at[0, pl.ds(tb, TAIL)],
                            pa_v.at[pl.ds(0, TAIL)])
            pltpu.sync_copy(p_hbm.at[1, pl.ds(tb, TAIL)],
                            pb_v.at[pl.ds(0, TAIL)])
            pltpu.sync_copy(z_hbm.at[pl.ds(tb, TAIL)],
                            znode_v.at[pl.ds(0, TAIL)])
            _post0_rows(pa_v, pb_v, znode_v, hbuf_v, bias, gi, be, TAIL)
            pltpu.sync_copy(hbuf_v.at[pl.ds(0, TAIL)],
                            h_hbm.at[c, pl.ds(tb, TAIL)])

        plsc.subcore_barrier()
        _emit_spmm(h_hbm.at[c], src_v, dst_v, w_v, acc, scr)
        plsc.subcore_barrier()
        _write_out(acc, out_hbm, c, s)

    return k(p, z, aff, ei, w, zeros)[0]


def _spmm_post1_sc(p, W1, aff, ei, w, zeros):
    @functools.partial(
        pl.kernel,
        mesh=_MESH,
        out_type=(jax.ShapeDtypeStruct((NC, N, H), jnp.float32),
                  jax.ShapeDtypeStruct((NC, N, H), jnp.float32)),
        scratch_types=(_EDGE_SCRATCH + [
            pltpu.VMEM((ROWS_T, H), jnp.float32),
            pltpu.VMEM((ROWS_T, H), jnp.float32),
            pltpu.VMEM((ROWS_T, H), jnp.float32),
            pltpu.VMEM((3, H), jnp.float32),
            pltpu.VMEM((H, H), jnp.float32),
        ] + _RING_SCRATCH),
        compiler_params=pltpu.CompilerParams(use_tc_tiling_on_sc=False),
    )
    def k(p_hbm, w1_hbm, aff_hbm, ei_hbm, w_hbm, zz_hbm, out_hbm, h_hbm,
          src_v, dst_v, w_v, acc, pa_v, pb_v, hbuf_v, aff_v, w1_v,
          *scr):
        c = lax.axis_index("c")
        s = lax.axis_index("s")
        _stage_edges(ei_hbm, w_hbm, zz_hbm, src_v, dst_v, w_v,
                     acc, c, s)
        rbase = s * ROWS_T
        pltpu.sync_copy(p_hbm.at[0, pl.ds(rbase, ROWS_T)], pa_v)
        pltpu.sync_copy(p_hbm.at[1, pl.ds(rbase, ROWS_T)], pb_v)
        pltpu.sync_copy(aff_hbm, aff_v)
        pltpu.sync_copy(w1_hbm, w1_v)
        bias, gi, be = aff_v[0], aff_v[1], aff_v[2]
        wrows = [w1_v[kk] for kk in range(H)]

        def matmul_rows(n_rows):
            def grp(g, carry):
                for r in range(L):
                    i = g * L + r
                    srow = pa_v[i] + pb_v[i]
                    t = bias
                    for kk in range(H):
                        t = t + _lane_bcast(srow, kk) * wrows[kk]
                    hbuf_v[i] = jnp.maximum(t * gi + be, 0.0)
                return carry

            lax.fori_loop(0, n_rows // L, grp, 0)

        matmul_rows(ROWS_T)
        pltpu.sync_copy(hbuf_v, h_hbm.at[c, pl.ds(rbase, ROWS_T)])

        @pl.when(s == 0)
        def _tail():
            tb = ROWS_T * NS
            pltpu.sync_copy(p_hbm.at[0, pl.ds(tb, TAIL)],
                            pa_v.at[pl.ds(0, TAIL)])
            pltpu.sync_copy(p_hbm.at[1, pl.ds(tb, TAIL)],
                            pb_v.at[pl.ds(0, TAIL)])
            matmul_rows(TAIL)
            pltpu.sync_copy(hbuf_v.at[pl.ds(0, TAIL)],
                            h_hbm.at[c, pl.ds(tb, TAIL)])

        plsc.subcore_barrier()
        _emit_spmm(h_hbm.at[c], src_v, dst_v, w_v, acc, scr)
        plsc.subcore_barrier()
        _write_out(acc, out_hbm, c, s)

    return k(p, W1, aff, ei, w, zeros)[0]


def _dense_in(x, M, W0):
    def body(x_ref, m_ref, w_ref, o_ref):
        o_ref[...] = jnp.dot(x_ref[...] * m_ref[...], w_ref[...],
                             preferred_element_type=jnp.float32)
    return pl.pallas_call(
        body, out_shape=jax.ShapeDtypeStruct((N, H), jnp.float32))(x, M, W0)


def _post2(p, W2, b2):
    def body(p_ref, w_ref, bias_ref, o_ref):
        t = jnp.dot(p_ref[0] + p_ref[1], w_ref[...],
                    preferred_element_type=jnp.float32) + bias_ref[...]
        m = jnp.max(t, axis=-1, keepdims=True)
        lse = jnp.log(jnp.sum(jnp.exp(t - m), axis=-1, keepdims=True)) + m
        o_ref[...] = t - lse
    return pl.pallas_call(
        body, out_shape=jax.ShapeDtypeStruct((N, C), jnp.float32))(
            p, W2, b2)


def kernel(x, edge_index, edge_weight, edge_indexZ, edge_weightZ, M, Z,
           W0, b0, gamma0, beta0, W1, b1, gamma1, beta1, W2, b2):
    eiZ = edge_indexZ.astype(jnp.int32)
    ei = edge_index.astype(jnp.int32)
    zeros = jnp.zeros((N, H), jnp.float32)
    aff0 = jnp.stack([b0, gamma0 * INV_BN, beta0])
    aff1 = jnp.stack([b1, gamma1 * INV_BN, beta1])

    y0 = _dense_in(x, M, W0)
    p0 = _spmm_sc(y0, eiZ, edge_weightZ, zeros)
    p1 = _spmm_post0_sc(p0, Z.ravel(), aff0, ei, edge_weight, zeros)
    p2 = _spmm_post1_sc(p1, W1, aff1, ei, edge_weight, zeros)
    return _post2(p2, W2, b2.reshape(1, C))


# speedup vs baseline: 1.0783x; 1.0783x over previous
import functools

import jax
import jax.numpy as jnp
from jax import lax
from jax.experimental import pallas as pl
from jax.experimental.pallas import tpu as pltpu
from jax.experimental.pallas import tpu_sc as plsc

N = 10000
E = 320000
F_IN = 128
H = 16
C = 40
INV_BN = 1.0 / (1.0 + 1e-5) ** 0.5

NC = 2
NS = 16
L = 16
NW = NC * NS
EPT = E // NW
K = 80
NCH = EPT // K
NBUF = 5
RND = NCH // NBUF
ROWS_T = 624
TAIL = N - ROWS_T * NS

_GDN = lax.GatherDimensionNumbers(
    offset_dims=(), collapsed_slice_dims=(0,), start_index_map=(0,))


def _lane_bcast(v, r):
    idx = jnp.full((L, 1), r, dtype=jnp.int32)
    return lax.gather(v, idx, _GDN, (1,),
                      mode=lax.GatherScatterMode.PROMISE_IN_BOUNDS)


_MESH = plsc.VectorSubcoreMesh(core_axis_name="c", subcore_axis_name="s")

_EDGE_SCRATCH = [
    pltpu.VMEM((EPT,), jnp.int32),
    pltpu.VMEM((EPT,), jnp.int32),
    pltpu.VMEM((EPT,), jnp.float32),
    pltpu.VMEM_SHARED((N, H), jnp.float32),
]
_RING_SCRATCH = ([pltpu.VMEM((K, L), jnp.float32)] * (2 * NBUF)
                 + [pltpu.SemaphoreType.DMA] * (2 * NBUF))


def _stage_edges(ei_hbm, w_hbm, z_hbm, src_v, dst_v, w_v, acc, c, s):
    ebase0 = (c * NS + s) * EPT
    pltpu.sync_copy(ei_hbm.at[0, pl.ds(ebase0, EPT)], src_v)
    pltpu.sync_copy(ei_hbm.at[1, pl.ds(ebase0, EPT)], dst_v)
    pltpu.sync_copy(w_hbm.at[pl.ds(ebase0, EPT)], w_v)
    rbase = s * ROWS_T
    pltpu.sync_copy(z_hbm.at[pl.ds(rbase, ROWS_T)],
                    acc.at[pl.ds(rbase, ROWS_T)])

    @pl.when(s == 0)
    def _zero_tail():
        pltpu.sync_copy(z_hbm.at[pl.ds(ROWS_T * NS, TAIL)],
                        acc.at[pl.ds(ROWS_T * NS, TAIL)])


def _emit_spmm(ysrc, src_v, dst_v, w_v, acc, scr):
    gbuf = scr[0:NBUF]
    sbuf = scr[NBUF:2 * NBUF]
    gsem = scr[2 * NBUF:3 * NBUF]
    ssem = scr[3 * NBUF:4 * NBUF]
    iota16 = lax.iota(jnp.int32, L)

    def g_start(jj, b):
        pltpu.async_copy(ysrc.at[src_v.at[pl.ds(jj * K, K)]],
                         gbuf[b], gsem[b])

    def g_wait(b):
        pltpu.make_async_copy(ysrc.at[src_v.at[pl.ds(0, K)]],
                              gbuf[b], gsem[b]).wait()

    def s_start(jj, b):
        for q in range(K // L):
            idx16 = dst_v[pl.ds(jj * K + q * L, L)]
            pltpu.async_copy(sbuf[b].at[pl.ds(q * L, L)], acc.at[idx16],
                             ssem[b], add=True)

    def s_wait(b):
        for q in range(K // L):
            pltpu.make_async_copy(sbuf[b].at[pl.ds(q * L, L)],
                                  acc.at[iota16], ssem[b]).wait()

    def compute(jj, b):
        for q in range(K // L):
            w16 = w_v[pl.ds(jj * K + q * L, L)]
            for r in range(L):
                e = q * L + r
                sbuf[b][e] = gbuf[b][e] * _lane_bcast(w16, r)

    for b in range(NBUF):
        g_start(b, b)

    def round_body(rr, carry):
        for b in range(NBUF):
            jj = rr * NBUF + b
            g_wait(b)

            @pl.when(rr > 0)
            def _drain():
                s_wait(b)

            compute(jj, b)
            s_start(jj, b)

            @pl.when(rr < RND - 1)
            def _prefetch():
                g_start(jj + NBUF, b)

        return carry

    lax.fori_loop(0, RND, round_body, 0)
    for b in range(NBUF):
        s_wait(b)


def _write_out(acc, out_hbm, c, s):
    rbase = s * ROWS_T
    pltpu.sync_copy(acc.at[pl.ds(rbase, ROWS_T)],
                    out_hbm.at[c, pl.ds(rbase, ROWS_T)])

    @pl.when(s == 0)
    def _write_tail():
        pltpu.sync_copy(acc.at[pl.ds(ROWS_T * NS, TAIL)],
                        out_hbm.at[c, pl.ds(ROWS_T * NS, TAIL)])


def _spmm_sc(y, ei, w, zeros):
    @functools.partial(
        pl.kernel,
        mesh=_MESH,
        out_type=jax.ShapeDtypeStruct((NC, N, H), jnp.float32),
        scratch_types=_EDGE_SCRATCH + _RING_SCRATCH,
        compiler_params=pltpu.CompilerParams(use_tc_tiling_on_sc=False),
    )
    def k(y_hbm, ei_hbm, w_hbm, z_hbm, out_hbm,
          src_v, dst_v, w_v, acc, *scr):
        c = lax.axis_index("c")
        s = lax.axis_index("s")
        _stage_edges(ei_hbm, w_hbm, z_hbm, src_v, dst_v, w_v,
                     acc, c, s)
        plsc.subcore_barrier()
        _emit_spmm(y_hbm, src_v, dst_v, w_v, acc, scr)
        plsc.subcore_barrier()
        _write_out(acc, out_hbm, c, s)

    return k(y, ei, w, zeros)


def _post0_rows(pa_v, pb_v, zz_v, hbuf_v, bias, gi, be, n_rows):
    def grp(g, carry):
        z16 = zz_v[pl.ds(g * L, L)]
        for r in range(L):
            i = g * L + r
            t = (pa_v[i] + pb_v[i]) * _lane_bcast(z16, r) + bias
            hbuf_v[i] = jnp.maximum(t * gi + be, 0.0)
        return carry

    lax.fori_loop(0, n_rows // L, grp, 0)


def _spmm_post0_sc(p, z, aff, ei, w, zeros):
    @functools.partial(
        pl.kernel,
        mesh=_MESH,
        out_type=jax.ShapeDtypeStruct((NC, N, H), jnp.float32),
        scratch_types=(_EDGE_SCRATCH + [
            pltpu.VMEM_SHARED((N, H), jnp.float32),
            pltpu.VMEM((ROWS_T, H), jnp.float32),
            pltpu.VMEM((ROWS_T, H), jnp.float32),
            pltpu.VMEM((ROWS_T,), jnp.float32),
            pltpu.VMEM((ROWS_T, H), jnp.float32),
            pltpu.VMEM((3, H), jnp.float32),
        ] + _RING_SCRATCH),
        compiler_params=pltpu.CompilerParams(use_tc_tiling_on_sc=False),
    )
    def k(p_hbm, z_hbm, aff_hbm, ei_hbm, w_hbm, zz_hbm, out_hbm,
          src_v, dst_v, w_v, acc, h_sh, pa_v, pb_v, znode_v, hbuf_v, aff_v,
          *scr):
        c = lax.axis_index("c")
        s = lax.axis_index("s")
        _stage_edges(ei_hbm, w_hbm, zz_hbm, src_v, dst_v, w_v,
                     acc, c, s)
        rbase = s * ROWS_T
        pltpu.sync_copy(p_hbm.at[0, pl.ds(rbase, ROWS_T)], pa_v)
        pltpu.sync_copy(p_hbm.at[1, pl.ds(rbase, ROWS_T)], pb_v)
        pltpu.sync_copy(z_hbm.at[pl.ds(rbase, ROWS_T)], znode_v)
        pltpu.sync_copy(aff_hbm, aff_v)
        bias, gi, be = aff_v[0], aff_v[1], aff_v[2]
        _post0_rows(pa_v, pb_v, znode_v, hbuf_v, bias, gi, be, ROWS_T)
        pltpu.sync_copy(hbuf_v, h_sh.at[pl.ds(rbase, ROWS_T)])

        @pl.when(s == 0)
        def _tail():
            tb = ROWS_T * NS
            pltpu.sync_copy(p_hbm.at[0, pl.ds(tb, TAIL)],
                            pa_v.at[pl.ds(0, TAIL)])
            pltpu.sync_copy(p_hbm.at[1, pl.ds(tb, TAIL)],
                            pb_v.at[pl.ds(0, TAIL)])
            pltpu.sync_copy(z_hbm.at[pl.ds(tb, TAIL)],
                            znode_v.at[pl.ds(0, TAIL)])
            _post0_rows(pa_v, pb_v, znode_v, hbuf_v, bias, gi, be, TAIL)
            pltpu.sync_copy(hbuf_v.at[pl.ds(0, TAIL)],
                            h_sh.at[pl.ds(tb, TAIL)])

        plsc.subcore_barrier()
        _emit_spmm(h_sh, src_v, dst_v, w_v, acc, scr)
        plsc.subcore_barrier()
        _write_out(acc, out_hbm, c, s)

    return k(p, z, aff, ei, w, zeros)


def _spmm_post1_sc(p, W1, aff, ei, w, zeros):
    @functools.partial(
        pl.kernel,
        mesh=_MESH,
        out_type=jax.ShapeDtypeStruct((NC, N, H), jnp.float32),
        scratch_types=(_EDGE_SCRATCH + [
            pltpu.VMEM_SHARED((N, H), jnp.float32),
            pltpu.VMEM((ROWS_T, H), jnp.float32),
            pltpu.VMEM((ROWS_T, H), jnp.float32),
            pltpu.VMEM((ROWS_T, H), jnp.float32),
            pltpu.VMEM((3, H), jnp.float32),
            pltpu.VMEM((H, H), jnp.float32),
        ] + _RING_SCRATCH),
        compiler_params=pltpu.CompilerParams(use_tc_tiling_on_sc=False),
    )
    def k(p_hbm, w1_hbm, aff_hbm, ei_hbm, w_hbm, zz_hbm, out_hbm,
          src_v, dst_v, w_v, acc, h_sh, pa_v, pb_v, hbuf_v, aff_v, w1_v,
          *scr):
        c = lax.axis_index("c")
        s = lax.axis_index("s")
        _stage_edges(ei_hbm, w_hbm, zz_hbm, src_v, dst_v, w_v,
                     acc, c, s)
        rbase = s * ROWS_T
        pltpu.sync_copy(p_hbm.at[0, pl.ds(rbase, ROWS_T)], pa_v)
        pltpu.sync_copy(p_hbm.at[1, pl.ds(rbase, ROWS_T)], pb_v)
        pltpu.sync_copy(aff_hbm, aff_v)
        pltpu.sync_copy(w1_hbm, w1_v)
        bias, gi, be = aff_v[0], aff_v[1], aff_v[2]
        wrows = [w1_v[kk] for kk in range(H)]

        def matmul_rows(n_rows):
            def grp(g, carry):
                for r in range(L):
                    i = g * L + r
                    srow = pa_v[i] + pb_v[i]
                    t = bias
                    for kk in range(H):
                        t = t + _lane_bcast(srow, kk) * wrows[kk]
                    hbuf_v[i] = jnp.maximum(t * gi + be, 0.0)
                return carry

            lax.fori_loop(0, n_rows // L, grp, 0)

        matmul_rows(ROWS_T)
        pltpu.sync_copy(hbuf_v, h_sh.at[pl.ds(rbase, ROWS_T)])

        @pl.when(s == 0)
        def _tail():
            tb = ROWS_T * NS
            pltpu.sync_copy(p_hbm.at[0, pl.ds(tb, TAIL)],
                            pa_v.at[pl.ds(0, TAIL)])
            pltpu.sync_copy(p_hbm.at[1, pl.ds(tb, TAIL)],
                            pb_v.at[pl.ds(0, TAIL)])
            matmul_rows(TAIL)
            pltpu.sync_copy(hbuf_v.at[pl.ds(0, TAIL)],
                            h_sh.at[pl.ds(tb, TAIL)])

        plsc.subcore_barrier()
        _emit_spmm(h_sh, src_v, dst_v, w_v, acc, scr)
        plsc.subcore_barrier()
        _write_out(acc, out_hbm, c, s)

    return k(p, W1, aff, ei, w, zeros)


def _dense_in(x, M, W0):
    BR = 1000

    def body(x_ref, m_ref, w_ref, o_ref):
        o_ref[...] = jnp.dot(x_ref[...] * m_ref[...], w_ref[...],
                             preferred_element_type=jnp.float32)

    return pl.pallas_call(
        body,
        grid=(N // BR,),
        in_specs=[pl.BlockSpec((BR, F_IN), lambda i: (i, 0)),
                  pl.BlockSpec((BR, 1), lambda i: (i, 0)),
                  pl.BlockSpec((F_IN, H), lambda i: (0, 0))],
        out_specs=pl.BlockSpec((BR, H), lambda i: (i, 0)),
        out_shape=jax.ShapeDtypeStruct((N, H), jnp.float32))(x, M, W0)


def _post2(p, W2, b2):
    def body(p_ref, w_ref, bias_ref, o_ref):
        t = jnp.dot(p_ref[0] + p_ref[1], w_ref[...],
                    preferred_element_type=jnp.float32) + bias_ref[...]
        m = jnp.max(t, axis=-1, keepdims=True)
        lse = jnp.log(jnp.sum(jnp.exp(t - m), axis=-1, keepdims=True)) + m
        o_ref[...] = t - lse
    return pl.pallas_call(
        body, out_shape=jax.ShapeDtypeStruct((N, C), jnp.float32))(
            p, W2, b2)


def kernel(x, edge_index, edge_weight, edge_indexZ, edge_weightZ, M, Z,
           W0, b0, gamma0, beta0, W1, b1, gamma1, beta1, W2, b2):
    eiZ = edge_indexZ.astype(jnp.int32)
    ei = edge_index.astype(jnp.int32)
    zeros = jnp.zeros((N, H), jnp.float32)
    aff0 = jnp.stack([b0, gamma0 * INV_BN, beta0])
    aff1 = jnp.stack([b1, gamma1 * INV_BN, beta1])

    y0 = _dense_in(x, M, W0)
    p0 = _spmm_sc(y0, eiZ, edge_weightZ, zeros)
    p1 = _spmm_post0_sc(p0, Z.ravel(), aff0, ei, edge_weight, zeros)
    p2 = _spmm_post1_sc(p1, W1, aff1, ei, edge_weight, zeros)
    return _post2(p2, W2, b2.reshape(1, C))


# speedup vs baseline: 1.1016x; 1.0216x over previous
import functools

import jax
import jax.numpy as jnp
from jax import lax
from jax.experimental import pallas as pl
from jax.experimental.pallas import tpu as pltpu
from jax.experimental.pallas import tpu_sc as plsc

N = 10000
E = 320000
F_IN = 128
H = 16
C = 40
INV_BN = 1.0 / (1.0 + 1e-5) ** 0.5

NC = 2
NS = 16
L = 16
NW = NC * NS
EPT = E // NW
K = 80
NCH = EPT // K
NBUF = 5
RND = NCH // NBUF
ROWS_T = 624
TAIL = N - ROWS_T * NS

_GDN = lax.GatherDimensionNumbers(
    offset_dims=(), collapsed_slice_dims=(0,), start_index_map=(0,))


def _lane_bcast(v, r):
    idx = jnp.full((L, 1), r, dtype=jnp.int32)
    return lax.gather(v, idx, _GDN, (1,),
                      mode=lax.GatherScatterMode.PROMISE_IN_BOUNDS)


_MESH = plsc.VectorSubcoreMesh(core_axis_name="c", subcore_axis_name="s")

_EDGE_SCRATCH = [
    pltpu.VMEM((EPT,), jnp.int32),
    pltpu.VMEM((EPT,), jnp.int32),
    pltpu.VMEM((EPT,), jnp.float32),
    pltpu.VMEM_SHARED((N, H), jnp.float32),
]
_RING_SCRATCH = ([pltpu.VMEM((K, L), jnp.float32)] * (2 * NBUF)
                 + [pltpu.SemaphoreType.DMA] * (2 * NBUF))


def _stage_edges(ei_hbm, w_hbm, z_hbm, src_v, dst_v, w_v, acc, c, s):
    ebase0 = (c * NS + s) * EPT
    pltpu.sync_copy(ei_hbm.at[0, pl.ds(ebase0, EPT)], src_v)
    pltpu.sync_copy(ei_hbm.at[1, pl.ds(ebase0, EPT)], dst_v)
    pltpu.sync_copy(w_hbm.at[pl.ds(ebase0, EPT)], w_v)
    rbase = s * ROWS_T
    pltpu.sync_copy(z_hbm.at[pl.ds(rbase, ROWS_T)],
                    acc.at[pl.ds(rbase, ROWS_T)])

    @pl.when(s == 0)
    def _zero_tail():
        pltpu.sync_copy(z_hbm.at[pl.ds(ROWS_T * NS, TAIL)],
                        acc.at[pl.ds(ROWS_T * NS, TAIL)])


def _emit_spmm(ysrc, src_v, dst_v, w_v, acc, scr):
    gbuf = scr[0:NBUF]
    sbuf = scr[NBUF:2 * NBUF]
    gsem = scr[2 * NBUF:3 * NBUF]
    ssem = scr[3 * NBUF:4 * NBUF]
    iota16 = lax.iota(jnp.int32, L)

    def g_start(jj, b):
        pltpu.async_copy(ysrc.at[src_v.at[pl.ds(jj * K, K)]],
                         gbuf[b], gsem[b])

    def g_wait(b):
        pltpu.make_async_copy(ysrc.at[src_v.at[pl.ds(0, K)]],
                              gbuf[b], gsem[b]).wait()

    def s_start(jj, b):
        for q in range(K // L):
            idx16 = dst_v[pl.ds(jj * K + q * L, L)]
            pltpu.async_copy(sbuf[b].at[pl.ds(q * L, L)], acc.at[idx16],
                             ssem[b], add=True)

    def s_wait(b):
        for q in range(K // L):
            pltpu.make_async_copy(sbuf[b].at[pl.ds(q * L, L)],
                                  acc.at[iota16], ssem[b]).wait()

    def compute(jj, b):
        for q in range(K // L):
            w16 = w_v[pl.ds(jj * K + q * L, L)]
            for r in range(L):
                e = q * L + r
                sbuf[b][e] = gbuf[b][e] * _lane_bcast(w16, r)

    for b in range(NBUF):
        g_start(b, b)

    def round_body(rr, carry):
        for b in range(NBUF):
            jj = rr * NBUF + b
            g_wait(b)

            @pl.when(rr > 0)
            def _drain():
                s_wait(b)

            compute(jj, b)
            s_start(jj, b)

            @pl.when(rr < RND - 1)
            def _prefetch():
                g_start(jj + NBUF, b)

        return carry

    lax.fori_loop(0, RND, round_body, 0)
    for b in range(NBUF):
        s_wait(b)


def _write_out(acc, out_hbm, c, s):
    rbase = s * ROWS_T
    pltpu.sync_copy(acc.at[pl.ds(rbase, ROWS_T)],
                    out_hbm.at[c, pl.ds(rbase, ROWS_T)])

    @pl.when(s == 0)
    def _write_tail():
        pltpu.sync_copy(acc.at[pl.ds(ROWS_T * NS, TAIL)],
                        out_hbm.at[c, pl.ds(ROWS_T * NS, TAIL)])


def _spmm_sc(y, ei, w, zeros):
    @functools.partial(
        pl.kernel,
        mesh=_MESH,
        out_type=jax.ShapeDtypeStruct((NC, N, H), jnp.float32),
        scratch_types=_EDGE_SCRATCH + _RING_SCRATCH,
        compiler_params=pltpu.CompilerParams(use_tc_tiling_on_sc=False),
    )
    def k(y_hbm, ei_hbm, w_hbm, z_hbm, out_hbm,
          src_v, dst_v, w_v, acc, *scr):
        c = lax.axis_index("c")
        s = lax.axis_index("s")
        _stage_edges(ei_hbm, w_hbm, z_hbm, src_v, dst_v, w_v,
                     acc, c, s)
        plsc.subcore_barrier()
        _emit_spmm(y_hbm, src_v, dst_v, w_v, acc, scr)
        plsc.subcore_barrier()
        _write_out(acc, out_hbm, c, s)

    return k(y, ei, w, zeros)


def _post0_rows(pa_v, pb_v, zz_v, hbuf_v, bias, gi, be, n_rows):
    def grp(g, carry):
        z16 = zz_v[pl.ds(g * L, L)]
        for r in range(L):
            i = g * L + r
            t = (pa_v[i] + pb_v[i]) * _lane_bcast(z16, r) + bias
            hbuf_v[i] = jnp.maximum(t * gi + be, 0.0)
        return carry

    lax.fori_loop(0, n_rows // L, grp, 0)


def _spmm_post0_sc(p, z, aff, ei, w, zeros):
    @functools.partial(
        pl.kernel,
        mesh=_MESH,
        out_type=jax.ShapeDtypeStruct((NC, N, H), jnp.float32),
        scratch_types=(_EDGE_SCRATCH + [
            pltpu.VMEM_SHARED((N, H), jnp.float32),
            pltpu.VMEM((ROWS_T, H), jnp.float32),
            pltpu.VMEM((ROWS_T, H), jnp.float32),
            pltpu.VMEM((ROWS_T,), jnp.float32),
            pltpu.VMEM((ROWS_T, H), jnp.float32),
            pltpu.VMEM((3, H), jnp.float32),
        ] + _RING_SCRATCH),
        compiler_params=pltpu.CompilerParams(use_tc_tiling_on_sc=False),
    )
    def k(p_hbm, z_hbm, aff_hbm, ei_hbm, w_hbm, zz_hbm, out_hbm,
          src_v, dst_v, w_v, acc, h_sh, pa_v, pb_v, znode_v, hbuf_v, aff_v,
          *scr):
        c = lax.axis_index("c")
        s = lax.axis_index("s")
        _stage_edges(ei_hbm, w_hbm, zz_hbm, src_v, dst_v, w_v,
                     acc, c, s)
        rbase = s * ROWS_T
        pltpu.sync_copy(p_hbm.at[0, pl.ds(rbase, ROWS_T)], pa_v)
        pltpu.sync_copy(p_hbm.at[1, pl.ds(rbase, ROWS_T)], pb_v)
        pltpu.sync_copy(z_hbm.at[pl.ds(rbase, ROWS_T)], znode_v)
        pltpu.sync_copy(aff_hbm, aff_v)
        bias, gi, be = aff_v[0], aff_v[1], aff_v[2]
        _post0_rows(pa_v, pb_v, znode_v, hbuf_v, bias, gi, be, ROWS_T)
        pltpu.sync_copy(hbuf_v, h_sh.at[pl.ds(rbase, ROWS_T)])

        @pl.when(s == 0)
        def _tail():
            tb = ROWS_T * NS
            pltpu.sync_copy(p_hbm.at[0, pl.ds(tb, TAIL)],
                            pa_v.at[pl.ds(0, TAIL)])
            pltpu.sync_copy(p_hbm.at[1, pl.ds(tb, TAIL)],
                            pb_v.at[pl.ds(0, TAIL)])
            pltpu.sync_copy(z_hbm.at[pl.ds(tb, TAIL)],
                            znode_v.at[pl.ds(0, TAIL)])
            _post0_rows(pa_v, pb_v, znode_v, hbuf_v, bias, gi, be, TAIL)
            pltpu.sync_copy(hbuf_v.at[pl.ds(0, TAIL)],
                            h_sh.at[pl.ds(tb, TAIL)])

        plsc.subcore_barrier()
        _emit_spmm(h_sh, src_v, dst_v, w_v, acc, scr)
        plsc.subcore_barrier()
        _write_out(acc, out_hbm, c, s)

    return k(p, z, aff, ei, w, zeros)


def _spmm_post1_sc(p, W1, aff, ei, w, zeros):
    @functools.partial(
        pl.kernel,
        mesh=_MESH,
        out_type=jax.ShapeDtypeStruct((NC, N, H), jnp.float32),
        scratch_types=(_EDGE_SCRATCH + [
            pltpu.VMEM_SHARED((N, H), jnp.float32),
            pltpu.VMEM((ROWS_T, H), jnp.float32),
            pltpu.VMEM((ROWS_T, H), jnp.float32),
            pltpu.VMEM((ROWS_T, H), jnp.float32),
            pltpu.VMEM((3, H), jnp.float32),
            pltpu.VMEM((H, H), jnp.float32),
        ] + _RING_SCRATCH),
        compiler_params=pltpu.CompilerParams(use_tc_tiling_on_sc=False),
    )
    def k(p_hbm, w1_hbm, aff_hbm, ei_hbm, w_hbm, zz_hbm, out_hbm,
          src_v, dst_v, w_v, acc, h_sh, pa_v, pb_v, hbuf_v, aff_v, w1_v,
          *scr):
        c = lax.axis_index("c")
        s = lax.axis_index("s")
        _stage_edges(ei_hbm, w_hbm, zz_hbm, src_v, dst_v, w_v,
                     acc, c, s)
        rbase = s * ROWS_T
        pltpu.sync_copy(p_hbm.at[0, pl.ds(rbase, ROWS_T)], pa_v)
        pltpu.sync_copy(p_hbm.at[1, pl.ds(rbase, ROWS_T)], pb_v)
        pltpu.sync_copy(aff_hbm, aff_v)
        pltpu.sync_copy(w1_hbm, w1_v)
        bias, gi, be = aff_v[0], aff_v[1], aff_v[2]
        wrows = [w1_v[kk] for kk in range(H)]

        def matmul_rows(n_rows):
            def grp(g, carry):
                for r in range(L):
                    i = g * L + r
                    srow = pa_v[i] + pb_v[i]
                    t = bias
                    for kk in range(H):
                        t = t + _lane_bcast(srow, kk) * wrows[kk]
                    hbuf_v[i] = jnp.maximum(t * gi + be, 0.0)
                return carry

            lax.fori_loop(0, n_rows // L, grp, 0)

        matmul_rows(ROWS_T)
        pltpu.sync_copy(hbuf_v, h_sh.at[pl.ds(rbase, ROWS_T)])

        @pl.when(s == 0)
        def _tail():
            tb = ROWS_T * NS
            pltpu.sync_copy(p_hbm.at[0, pl.ds(tb, TAIL)],
                            pa_v.at[pl.ds(0, TAIL)])
            pltpu.sync_copy(p_hbm.at[1, pl.ds(tb, TAIL)],
                            pb_v.at[pl.ds(0, TAIL)])
            matmul_rows(TAIL)
            pltpu.sync_copy(hbuf_v.at[pl.ds(0, TAIL)],
                            h_sh.at[pl.ds(tb, TAIL)])

        plsc.subcore_barrier()
        _emit_spmm(h_sh, src_v, dst_v, w_v, acc, scr)
        plsc.subcore_barrier()
        _write_out(acc, out_hbm, c, s)

    return k(p, W1, aff, ei, w, zeros)


def _dense_in(x, M, W0):
    def body(x_ref, m_ref, w_ref, o_ref):
        o_ref[...] = jnp.dot(x_ref[...] * m_ref[...], w_ref[...],
                             preferred_element_type=jnp.float32)
    return pl.pallas_call(
        body, out_shape=jax.ShapeDtypeStruct((N, H), jnp.float32))(x, M, W0)


def _post2(p, W2, b2):
    def body(p_ref, w_ref, bias_ref, o_ref):
        t = jnp.dot(p_ref[0] + p_ref[1], w_ref[...],
                    preferred_element_type=jnp.float32) + bias_ref[...]
        m = jnp.max(t, axis=-1, keepdims=True)
        lse = jnp.log(jnp.sum(jnp.exp(t - m), axis=-1, keepdims=True)) + m
        o_ref[...] = t - lse
    return pl.pallas_call(
        body, out_shape=jax.ShapeDtypeStruct((N, C), jnp.float32))(
            p, W2, b2)


def kernel(x, edge_index, edge_weight, edge_indexZ, edge_weightZ, M, Z,
           W0, b0, gamma0, beta0, W1, b1, gamma1, beta1, W2, b2):
    eiZ = edge_indexZ.astype(jnp.int32)
    ei = edge_index.astype(jnp.int32)
    zeros = jnp.zeros((N, H), jnp.float32)
    aff0 = jnp.stack([b0, gamma0 * INV_BN, beta0])
    aff1 = jnp.stack([b1, gamma1 * INV_BN, beta1])

    y0 = _dense_in(x, M, W0)
    p0 = _spmm_sc(y0, eiZ, edge_weightZ, zeros)
    p1 = _spmm_post0_sc(p0, Z.ravel(), aff0, ei, edge_weight, zeros)
    p2 = _spmm_post1_sc(p1, W1, aff1, ei, edge_weight, zeros)
    return _post2(p2, W2, b2.reshape(1, C))
